# baseline (speedup 1.0000x reference)
import functools

import jax
import jax.numpy as jnp
from jax.experimental import pallas as pl
from jax.experimental.pallas import tpu as pltpu
from jax.experimental.pallas import tpu_sc as plsc

N = 2048
D = 768
H = 768
E = 8

BE = 256
PADN = 6144
NBLK = PADN // BE
BG = 256
NBG = N // BG
NC, NS = 2, 16
NW = NC * NS
BPW = N // NW


def _top2(logits):
    neg_inf = jnp.float32(-jnp.inf)
    iota = jax.lax.broadcasted_iota(jnp.int32, logits.shape, 1)
    m1 = jnp.max(logits, axis=-1, keepdims=True)
    eq1 = logits == m1
    i1 = jnp.min(jnp.where(eq1, iota, E), axis=-1, keepdims=True)
    first1 = iota == i1
    l2 = jnp.where(first1, neg_inf, logits)
    m2 = jnp.max(l2, axis=-1, keepdims=True)
    eq2 = l2 == m2
    i2 = jnp.min(jnp.where(eq2, iota, E), axis=-1, keepdims=True)
    first2 = iota == i2
    sel = first1 | first2
    wsel = jnp.where(sel, jnp.exp(logits - m1), 0.0)
    wsum = jnp.sum(wsel, axis=-1, keepdims=True)
    return first1, first2, sel, wsel, wsum


def _gate_body(x_ref, wg_ref, bg_ref, d01_ref, w01_ref, eid_ref,
               counts_s, po_s, run_s):
    p = pl.program_id(0)
    i = pl.program_id(1)
    xb = x_ref[...]
    logits = jnp.dot(xb, wg_ref[...], preferred_element_type=jnp.float32)
    logits = logits + bg_ref[...]
    first1, first2, sel, wsel, wsum = _top2(logits)
    sel_f = sel.astype(jnp.float32)

    @pl.when((p == 0) & (i == 0))
    def _():
        counts_s[...] = jnp.zeros((1, E), jnp.float32)

    @pl.when(p == 0)
    def _():
        counts_s[...] += jnp.sum(sel_f, axis=0, keepdims=True)

    @pl.when((p == 1) & (i == 0))
    def _():
        c = counts_s[...]
        cpad = jnp.floor((c + (BE - 1)) / BE) * BE
        lt = (jax.lax.broadcasted_iota(jnp.int32, (E, E), 0)
              < jax.lax.broadcasted_iota(jnp.int32, (E, E), 1)
              ).astype(jnp.bfloat16)
        po_s[...] = jnp.dot(cpad.astype(jnp.bfloat16), lt,
                            preferred_element_type=jnp.float32)
        run_s[...] = jnp.zeros((1, E), jnp.float32)

    @pl.when(p == 1)
    def _():
        po = po_s[...]
        run = run_s[...]
        ltn = (jax.lax.broadcasted_iota(jnp.int32, (BG, BG), 1)
               < jax.lax.broadcasted_iota(jnp.int32, (BG, BG), 0)
               ).astype(jnp.bfloat16)
        pos = run + jnp.dot(ltn, sel_f.astype(jnp.bfloat16),
                            preferred_element_type=jnp.float32)
        run_s[...] = run + jnp.sum(sel_f, axis=0, keepdims=True)
        dest = po + pos
        d0 = jnp.sum(jnp.where(first1, dest, 0.0), axis=-1, keepdims=True)
        d1 = jnp.sum(jnp.where(first2, dest, 0.0), axis=-1, keepdims=True)
        w0 = jnp.sum(jnp.where(first1, wsel, 0.0), axis=-1, keepdims=True) / wsum
        w1 = jnp.sum(jnp.where(first2, wsel, 0.0), axis=-1, keepdims=True) / wsum
        col = jax.lax.broadcasted_iota(jnp.int32, (BG, E), 1)
        d01_ref[...] = ((col == 0) * d0.astype(jnp.int32)
                        + (col == 1) * d1.astype(jnp.int32))
        w01_ref[...] = (jnp.where(col == 0, w0, 0.0)
                        + jnp.where(col == 1, w1, 0.0))
        bst = jnp.float32(BE) * jax.lax.broadcasted_iota(
            jnp.int32, (E, NBLK), 1).astype(jnp.float32)
        m = (jnp.reshape(po, (E, 1)) <= bst).astype(jnp.int32)
        eid_ref[...] = jnp.sum(m, axis=0, keepdims=True) - 1


def _gate(x, Wg, bg):
    return pl.pallas_call(
        _gate_body,
        grid=(2, NBG),
        in_specs=[
            pl.BlockSpec((BG, D), lambda p, i: (i, 0)),
            pl.BlockSpec((D, E), lambda p, i: (0, 0)),
            pl.BlockSpec((1, E), lambda p, i: (0, 0)),
        ],
        out_specs=[
            pl.BlockSpec((BG, E), lambda p, i: (i, 0)),
            pl.BlockSpec((BG, E), lambda p, i: (i, 0)),
            pl.BlockSpec((1, NBLK), lambda p, i: (0, 0)),
        ],
        out_shape=[
            jax.ShapeDtypeStruct((N, E), jnp.int32),
            jax.ShapeDtypeStruct((N, E), jnp.float32),
            jax.ShapeDtypeStruct((1, NBLK), jnp.int32),
        ],
        scratch_shapes=[
            pltpu.VMEM((1, E), jnp.float32),
            pltpu.VMEM((1, E), jnp.float32),
            pltpu.VMEM((1, E), jnp.float32),
        ],
    )(x, Wg, bg.reshape(1, E))


def _sc_dispatch(x, d0, d1):
    mesh = plsc.VectorSubcoreMesh(core_axis_name="c", subcore_axis_name="s")

    @functools.partial(
        pl.kernel, mesh=mesh,
        out_type=jax.ShapeDtypeStruct((PADN, D), jnp.float32),
        scratch_types=[
            pltpu.VMEM((BPW,), jnp.int32),
            pltpu.VMEM((BPW,), jnp.int32),
            pltpu.VMEM((BPW, D), jnp.float32),
            pltpu.SemaphoreType.DMA,
        ],
    )
    def disp(x_hbm, d0_hbm, d1_hbm, xr_hbm, i0_v, i1_v, rows_v, sem):
        wid = jax.lax.axis_index("s") * NC + jax.lax.axis_index("c")
        base = wid * BPW
        pltpu.sync_copy(d0_hbm.at[pl.ds(base, BPW)], i0_v)
        pltpu.sync_copy(d1_hbm.at[pl.ds(base, BPW)], i1_v)
        pltpu.sync_copy(x_hbm.at[pl.ds(base, BPW)], rows_v)
        pltpu.async_copy(rows_v, xr_hbm.at[i0_v], sem).wait()
        pltpu.async_copy(rows_v, xr_hbm.at[i1_v], sem).wait()

    return disp(x, d0, d1)


def _sc_gather(y, d0, d1):
    mesh = plsc.VectorSubcoreMesh(core_axis_name="c", subcore_axis_name="s")

    @functools.partial(
        pl.kernel, mesh=mesh,
        out_type=[jax.ShapeDtypeStruct((N, H), jnp.float32),
                  jax.ShapeDtypeStruct((N, H), jnp.float32)],
        scratch_types=[
            pltpu.VMEM((BPW,), jnp.int32),
            pltpu.VMEM((BPW, H), jnp.float32),
            pltpu.SemaphoreType.DMA,
        ],
    )
    def gath(y_hbm, d0_hbm, d1_hbm, o0_hbm, o1_hbm, i_v, rows_v, sem):
        wid = jax.lax.axis_index("s") * NC + jax.lax.axis_index("c")
        base = wid * BPW
        pltpu.sync_copy(d0_hbm.at[pl.ds(base, BPW)], i_v)
        pltpu.async_copy(y_hbm.at[i_v], rows_v, sem).wait()
        pltpu.sync_copy(rows_v, o0_hbm.at[pl.ds(base, BPW)])
        pltpu.sync_copy(d1_hbm.at[pl.ds(base, BPW)], i_v)
        pltpu.async_copy(y_hbm.at[i_v], rows_v, sem).wait()
        pltpu.sync_copy(rows_v, o1_hbm.at[pl.ds(base, BPW)])

    return gath(y, d0, d1)


def _expert_body(eid_ref, xr_ref, w1_ref, b1_ref, g1_ref, be1_ref,
                 w2_ref, b2_ref, y_ref):
    xb16 = xr_ref[...].astype(jnp.bfloat16)
    h = jnp.dot(xb16, w1_ref[0], preferred_element_type=jnp.float32)
    h = h + b1_ref[0]
    mu = jnp.mean(h, axis=-1, keepdims=True)
    var = jnp.mean((h - mu) ** 2, axis=-1, keepdims=True)
    h = (h - mu) * jax.lax.rsqrt(var + 1e-5)
    h = h * g1_ref[0] + be1_ref[0]
    h = h * 0.5 * (1.0 + jax.lax.erf(h * jnp.float32(0.7071067811865476)))
    y = jnp.dot(h.astype(jnp.bfloat16), w2_ref[0],
                preferred_element_type=jnp.float32)
    y_ref[...] = y + b2_ref[0]


def _expert(eid, xr, w1b, b1, g1, be1, w2b, b2):
    grid_spec = pltpu.PrefetchScalarGridSpec(
        num_scalar_prefetch=1,
        grid=(NBLK,),
        in_specs=[
            pl.BlockSpec((BE, D), lambda i, eid_ref: (i, 0)),
            pl.BlockSpec((1, D, H), lambda i, eid_ref: (eid_ref[i], 0, 0)),
            pl.BlockSpec((1, 1, H), lambda i, eid_ref: (eid_ref[i], 0, 0)),
            pl.BlockSpec((1, 1, H), lambda i, eid_ref: (eid_ref[i], 0, 0)),
            pl.BlockSpec((1, 1, H), lambda i, eid_ref: (eid_ref[i], 0, 0)),
            pl.BlockSpec((1, D, H), lambda i, eid_ref: (eid_ref[i], 0, 0)),
            pl.BlockSpec((1, 1, H), lambda i, eid_ref: (eid_ref[i], 0, 0)),
        ],
        out_specs=pl.BlockSpec((BE, H), lambda i, eid_ref: (i, 0)),
    )
    return pl.pallas_call(
        _expert_body,
        grid_spec=grid_spec,
        out_shape=jax.ShapeDtypeStruct((PADN, H), jnp.float32),
    )(eid, xr, w1b, b1.reshape(E, 1, H), g1.reshape(E, 1, H),
      be1.reshape(E, 1, H), w2b, b2.reshape(E, 1, H))


def _combine_body(yg0_ref, yg1_ref, w01_ref, out_ref):
    w = w01_ref[...]
    out_ref[...] = yg0_ref[...] * w[:, 0:1] + yg1_ref[...] * w[:, 1:2]


def _combine(yg0, yg1, w01):
    return pl.pallas_call(
        _combine_body,
        grid=(NBG,),
        in_specs=[
            pl.BlockSpec((BG, H), lambda i: (i, 0)),
            pl.BlockSpec((BG, H), lambda i: (i, 0)),
            pl.BlockSpec((BG, E), lambda i: (i, 0)),
        ],
        out_specs=pl.BlockSpec((BG, H), lambda i: (i, 0)),
        out_shape=jax.ShapeDtypeStruct((N, H), jnp.float32),
    )(yg0, yg1, w01)


def kernel(x, Wg, bg, W1, b1, g1, be1, W2, b2):
    d01, w01, eidm = _gate(x, Wg, bg)
    d0 = d01[:, 0]
    d1 = d01[:, 1]
    xr = _sc_dispatch(x, d0, d1)
    w1b = W1.astype(jnp.bfloat16)
    w2b = W2.astype(jnp.bfloat16)
    eid = eidm.reshape(NBLK)
    y = _expert(eid, xr, w1b, b1, g1, be1, w2b, b2)
    yg0, yg1 = _sc_gather(y, d0, d1)
    return _combine(yg0, yg1, w01)
